# baseline (speedup 1.0000x reference)
import functools

import jax
import jax.numpy as jnp
from jax import lax
from jax.experimental import pallas as pl
from jax.experimental.pallas import tpu as pltpu
from jax.experimental.pallas import tpu_sc as plsc

BATCH = 4096
HIST = 200
D = 32
VOCAB = 1000000

NC = 2
NS = 16
NW = NC * NS
NTC = VOCAB // 128
LIN_ROWS = VOCAB // 4
Q, R = divmod(NTC, NW)


@jax.jit
def _impl(tag_ids, table):
    mesh = plsc.VectorSubcoreMesh(core_axis_name="c", subcore_axis_name="s")

    @functools.partial(
        pl.kernel,
        mesh=mesh,
        out_type=jax.ShapeDtypeStruct((LIN_ROWS, 128), jnp.float32),
        scratch_types=[
            pltpu.VMEM((32, 128), jnp.float32),
            pltpu.VMEM((32, 128), jnp.float32),
        ],
        compiler_params=pltpu.CompilerParams(use_tc_tiling_on_sc=True, needs_layout_passes=False),
    )
    def transpose_k(table_hbm, tail_hbm, lin_hbm, in_v, tr_v):
        wid = lax.axis_index("s") * NC + lax.axis_index("c")
        start = jnp.where(wid < R, wid * (Q + 1), R * (Q + 1) + (wid - R) * Q)
        cnt = jnp.where(wid < R, Q + 1, Q)
        iota = lax.iota(jnp.int32, 16)

        def do_block(nrows):
            for i in range(nrows):
                for g in range(4):
                    for hh in range(2):
                        vec = plsc.load_gather(
                            in_v,
                            [iota + 16 * hh, jnp.full((16,), 4 * i + g, jnp.int32)],
                        )
                        tr_v[i, pl.ds(32 * g + 16 * hh, 16)] = vec

        def body(j, carry):
            tc = start + j
            pltpu.sync_copy(table_hbm.at[:, pl.ds(tc * 128, 128)], in_v)
            do_block(32)
            pltpu.sync_copy(tr_v, lin_hbm.at[pl.ds(tc * 32, 32)])
            return carry

        lax.fori_loop(0, cnt, body, 0)

        @pl.when(wid == NW - 1)
        def _():
            pltpu.sync_copy(tail_hbm, in_v.at[pl.ds(0, 16)])
            pltpu.sync_copy(
                in_v.at[pl.ds(0, 16)], lin_hbm.at[pl.ds(NTC * 32, 16)]
            )

    @functools.partial(
        pl.kernel,
        mesh=mesh,
        out_type=jax.ShapeDtypeStruct((HIST, 4, NW, 8, 128), jnp.float32),
        scratch_types=[
            pltpu.VMEM((25, 8, 128), jnp.int32),
            pltpu.VMEM((128, 32), jnp.float32),
            pltpu.VMEM((4, 8, 128), jnp.float32),
            pltpu.SemaphoreType.DMA,
        ],
        compiler_params=pltpu.CompilerParams(
            use_tc_tiling_on_sc=False, needs_layout_passes=False
        ),
    )
    def gather_k(idx_hbm, lin_hbm, out_hbm, idx_v, rows_v, tr_v, sem):
        wid = lax.axis_index("s") * NC + lax.axis_index("c")
        pltpu.sync_copy(idx_hbm.at[:, wid], idx_v)
        iota = lax.iota(jnp.int32, 16)

        def body(h, carry):
            hr = h // 8
            s = h % 8
            pltpu.async_copy(lin_hbm.at[idx_v.at[hr, s]], rows_v, sem).wait()
            for tr in range(4):
                for sv in range(8):
                    for q in range(8):
                        vec = plsc.load_gather(
                            rows_v,
                            [iota + 16 * q, jnp.full((16,), 8 * tr + sv, jnp.int32)],
                        )
                        tr_v[tr, sv, pl.ds(16 * q, 16)] = vec
            pltpu.sync_copy(tr_v, out_hbm.at[h, :, wid])
            return carry

        lax.fori_loop(0, HIST, body, 0)

    tail_lin = table[NTC * 128 :].reshape(16, 128)
    lin = transpose_k(table.T, tail_lin)
    idx_view = (
        tag_ids.astype(jnp.int32).T.reshape(25, 8, NW, 128).transpose(0, 2, 1, 3)
    )
    o = gather_k(idx_view, lin.reshape(VOCAB, 32))
    return o.transpose(2, 4, 0, 1, 3).reshape(BATCH, HIST, D)


def kernel(tag_ids, embedding_table):
    return _impl(tag_ids, embedding_table)


# speedup vs baseline: 1.1219x; 1.1219x over previous
import functools

import jax
import jax.numpy as jnp
from jax import lax
from jax.experimental import pallas as pl
from jax.experimental.pallas import tpu as pltpu
from jax.experimental.pallas import tpu_sc as plsc

BATCH = 4096
HIST = 200
D = 32
VOCAB = 1000000

NC = 2
NS = 16
NW = NC * NS
NTC = VOCAB // 128
LIN_ROWS = VOCAB // 4

NG = NTC // 2
GQ, GR = divmod(NG, NW)

NHB = HIST // 2


@jax.jit
def _impl(tag_ids, table):
    mesh = plsc.VectorSubcoreMesh(core_axis_name="c", subcore_axis_name="s")

    @functools.partial(
        pl.kernel,
        mesh=mesh,
        out_type=jax.ShapeDtypeStruct((LIN_ROWS, 128), jnp.float32),
        scratch_types=[
            pltpu.VMEM((2, 32, 256), jnp.float32),
            pltpu.VMEM((2, 64, 128), jnp.float32),
            pltpu.SemaphoreType.DMA,
            pltpu.SemaphoreType.DMA,
            pltpu.SemaphoreType.DMA,
            pltpu.SemaphoreType.DMA,
        ],
        compiler_params=pltpu.CompilerParams(
            use_tc_tiling_on_sc=True, needs_layout_passes=False
        ),
    )
    def transpose_k(table_hbm, tail_hbm, lin_hbm, in_v, tr_v, gs0, gs1, ws0, ws1):
        wid = lax.axis_index("s") * NC + lax.axis_index("c")
        start = jnp.where(wid < GR, wid * (GQ + 1), GR * (GQ + 1) + (wid - GR) * GQ)
        cnt = jnp.where(wid < GR, GQ + 1, GQ)
        it16 = lax.iota(jnp.int32, 16)

        def fire_in(j, b, sem):
            tc = (start + j) * 2
            pltpu.async_copy(
                table_hbm.at[:, pl.ds(tc * 128, 256)], in_v.at[b], sem
            )

        def wait_in(b, sem):
            pltpu.make_async_copy(
                table_hbm.at[:, pl.ds(0, 256)], in_v.at[b], sem
            ).wait()

        def do_transpose(b):
            src = in_v.at[b]
            dst = tr_v.at[b]
            for i in range(64):
                for q in range(8):
                    row = it16 + 16 * (q % 2)
                    col = jnp.full((16,), 4 * i + q // 2, jnp.int32)
                    vec = plsc.load_gather(src, [row, col])
                    dst[i, pl.ds(16 * q, 16)] = vec

        def fire_out(j, b, sem):
            pltpu.async_copy(
                tr_v.at[b], lin_hbm.at[pl.ds((start + j) * 64, 64)], sem
            )

        def wait_out(b, sem):
            pltpu.make_async_copy(
                tr_v.at[b], lin_hbm.at[pl.ds(0, 64)], sem
            ).wait()

        fire_in(0, 0, gs0)

        def body(j, carry):
            b = lax.rem(j, 2)
            nb = 1 - b

            @pl.when(j + 1 < cnt)
            def _():
                @pl.when(b == 0)
                def _():
                    fire_in(j + 1, nb, gs1)

                @pl.when(b == 1)
                def _():
                    fire_in(j + 1, nb, gs0)

            @pl.when(b == 0)
            def _():
                wait_in(b, gs0)

            @pl.when(b == 1)
            def _():
                wait_in(b, gs1)

            @pl.when(j >= 2)
            def _():
                @pl.when(b == 0)
                def _():
                    wait_out(b, ws0)

                @pl.when(b == 1)
                def _():
                    wait_out(b, ws1)

            do_transpose(b)

            @pl.when(b == 0)
            def _():
                fire_out(j, b, ws0)

            @pl.when(b == 1)
            def _():
                fire_out(j, b, ws1)

            return carry

        lax.fori_loop(0, cnt, body, 0)

        @pl.when(lax.rem(cnt, 2) == 0)
        def _():
            wait_out(0, ws0)
            wait_out(1, ws1)

        @pl.when(lax.rem(cnt, 2) == 1)
        def _():
            wait_out(0, ws0)

            @pl.when(cnt >= 2)
            def _():
                wait_out(1, ws1)

        @pl.when(wid == NW - 1)
        def _():
            pltpu.sync_copy(tail_hbm, tr_v.at[0, pl.ds(0, 16)])
            pltpu.sync_copy(
                tr_v.at[0, pl.ds(0, 16)], lin_hbm.at[pl.ds(NTC * 32, 16)]
            )

    @functools.partial(
        pl.kernel,
        mesh=mesh,
        out_type=jax.ShapeDtypeStruct((HIST, 4, NW, 8, 128), jnp.float32),
        scratch_types=[
            pltpu.VMEM((25, 8, 128), jnp.int32),
            pltpu.VMEM((2, 256, 32), jnp.float32),
            pltpu.VMEM((2, 2, 4, 8, 128), jnp.float32),
            pltpu.SemaphoreType.DMA,
            pltpu.SemaphoreType.DMA,
            pltpu.SemaphoreType.DMA,
            pltpu.SemaphoreType.DMA,
        ],
        compiler_params=pltpu.CompilerParams(
            use_tc_tiling_on_sc=False, needs_layout_passes=False
        ),
    )
    def gather_k(idx_hbm, lin_hbm, out_hbm, idx_v, rw_v, tr_v, gs0, gs1, ws0, ws1):
        wid = lax.axis_index("s") * NC + lax.axis_index("c")
        pltpu.sync_copy(idx_hbm.at[:, wid], idx_v)
        it16 = lax.iota(jnp.int32, 16)

        def fire_gather(jb, b, sem):
            for t in range(2):
                h = 2 * jb + t
                pltpu.async_copy(
                    lin_hbm.at[idx_v.at[h // 8, h % 8]],
                    rw_v.at[b, pl.ds(128 * t, 128)],
                    sem,
                )

        def wait_gather(b, sem):
            for t in range(2):
                pltpu.make_async_copy(
                    lin_hbm.at[idx_v.at[0, 0]],
                    rw_v.at[b, pl.ds(128 * t, 128)],
                    sem,
                ).wait()

        def do_transpose(b):
            src = rw_v.at[b]
            dst = tr_v.at[b]
            for t in range(2):
                for tr in range(4):
                    for sv in range(8):
                        for q in range(8):
                            row = it16 + (128 * t + 16 * q)
                            col = jnp.full((16,), 8 * tr + sv, jnp.int32)
                            vec = plsc.load_gather(src, [row, col])
                            dst[t, tr, sv, pl.ds(16 * q, 16)] = vec

        def fire_out(jb, b, sem):
            pltpu.async_copy(
                tr_v.at[b], out_hbm.at[pl.ds(2 * jb, 2), :, wid], sem
            )

        def wait_out(b, sem):
            pltpu.make_async_copy(
                tr_v.at[b], out_hbm.at[pl.ds(0, 2), :, wid], sem
            ).wait()

        fire_gather(0, 0, gs0)

        def body(jb, carry):
            b = lax.rem(jb, 2)
            nb = 1 - b

            @pl.when(jb + 1 < NHB)
            def _():
                @pl.when(b == 0)
                def _():
                    fire_gather(jb + 1, nb, gs1)

                @pl.when(b == 1)
                def _():
                    fire_gather(jb + 1, nb, gs0)

            @pl.when(b == 0)
            def _():
                wait_gather(b, gs0)

            @pl.when(b == 1)
            def _():
                wait_gather(b, gs1)

            @pl.when(jb >= 2)
            def _():
                @pl.when(b == 0)
                def _():
                    wait_out(b, ws0)

                @pl.when(b == 1)
                def _():
                    wait_out(b, ws1)

            do_transpose(b)

            @pl.when(b == 0)
            def _():
                fire_out(jb, b, ws0)

            @pl.when(b == 1)
            def _():
                fire_out(jb, b, ws1)

            return carry

        lax.fori_loop(0, NHB, body, 0)
        wait_out(0, ws0)
        wait_out(1, ws1)

    tail_lin = table[NTC * 128 :].reshape(16, 128)
    lin = transpose_k(table.T, tail_lin)
    idx_view = (
        tag_ids.astype(jnp.int32).T.reshape(25, 8, NW, 128).transpose(0, 2, 1, 3)
    )
    o = gather_k(idx_view, lin.reshape(VOCAB, 32))
    return o.transpose(2, 4, 0, 1, 3).reshape(BATCH, HIST, D)


def kernel(tag_ids, embedding_table):
    return _impl(tag_ids, embedding_table)


# speedup vs baseline: 1.3010x; 1.1597x over previous
import functools

import jax
import jax.numpy as jnp
from jax import lax
from jax.experimental import pallas as pl
from jax.experimental.pallas import tpu as pltpu
from jax.experimental.pallas import tpu_sc as plsc

BATCH = 4096
HIST = 200
D = 32
VOCAB = 1000000

NC = 2
NS = 16
NW = NC * NS
NTC = VOCAB // 128
LIN_ROWS = VOCAB // 4

NG = NTC // 2
GQ, GR = divmod(NG, NW)

NHB = HIST // 2


@jax.jit
def _impl(tag_ids, table):
    mesh = plsc.VectorSubcoreMesh(core_axis_name="c", subcore_axis_name="s")

    @functools.partial(
        pl.kernel,
        mesh=mesh,
        out_type=jax.ShapeDtypeStruct((LIN_ROWS, 128), jnp.float32),
        scratch_types=[
            pltpu.VMEM((2, 64, 128), jnp.float32),
            pltpu.VMEM((2, 64, 128), jnp.float32),
            pltpu.SemaphoreType.DMA,
            pltpu.SemaphoreType.DMA,
            pltpu.SemaphoreType.DMA,
            pltpu.SemaphoreType.DMA,
        ],
        compiler_params=pltpu.CompilerParams(
            use_tc_tiling_on_sc=True, needs_layout_passes=False
        ),
    )
    def transpose_k(table_hbm, tail_hbm, lin_hbm, in_v, tr_v, gs0, gs1, ws0, ws1):
        wid = lax.axis_index("s") * NC + lax.axis_index("c")
        start = jnp.where(wid < GR, wid * (GQ + 1), GR * (GQ + 1) + (wid - GR) * GQ)
        cnt = jnp.where(wid < GR, GQ + 1, GQ)
        it16 = lax.iota(jnp.int32, 16)

        def fire_in(j, b, sem):
            tc = (start + j) * 2
            pltpu.async_copy(
                table_hbm.at[:, pl.ds(tc * 128, 128)],
                in_v.at[b, pl.ds(0, 32)], sem,
            )
            pltpu.async_copy(
                table_hbm.at[:, pl.ds((tc + 1) * 128, 128)],
                in_v.at[b, pl.ds(32, 32)], sem,
            )

        def wait_in(b, sem):
            for half in range(2):
                pltpu.make_async_copy(
                    table_hbm.at[:, pl.ds(0, 128)],
                    in_v.at[b, pl.ds(32 * half, 32)], sem,
                ).wait()

        def do_transpose(b):
            src = in_v.at[b]
            dst = tr_v.at[b]
            for i in range(64):
                for q in range(8):
                    rho = 4 * i + q // 2
                    row = it16 + (32 * (rho // 128) + 16 * (q % 2))
                    col = jnp.full((16,), rho % 128, jnp.int32)
                    vec = plsc.load_gather(src, [row, col])
                    plsc.store_scatter(
                        dst, [jnp.full((16,), i, jnp.int32), it16 + 16 * q], vec
                    )

        def fire_out(j, b, sem):
            pltpu.async_copy(
                tr_v.at[b], lin_hbm.at[pl.ds((start + j) * 64, 64)], sem
            )

        def wait_out(b, sem):
            pltpu.make_async_copy(
                tr_v.at[b], lin_hbm.at[pl.ds(0, 64)], sem
            ).wait()

        fire_in(0, 0, gs0)

        def body(j, carry):
            b = lax.rem(j, 2)
            nb = 1 - b

            @pl.when(j + 1 < cnt)
            def _():
                @pl.when(b == 0)
                def _():
                    fire_in(j + 1, nb, gs1)

                @pl.when(b == 1)
                def _():
                    fire_in(j + 1, nb, gs0)

            @pl.when(b == 0)
            def _():
                wait_in(b, gs0)

            @pl.when(b == 1)
            def _():
                wait_in(b, gs1)

            @pl.when(j >= 2)
            def _():
                @pl.when(b == 0)
                def _():
                    wait_out(b, ws0)

                @pl.when(b == 1)
                def _():
                    wait_out(b, ws1)

            do_transpose(b)

            @pl.when(b == 0)
            def _():
                fire_out(j, b, ws0)

            @pl.when(b == 1)
            def _():
                fire_out(j, b, ws1)

            return carry

        lax.fori_loop(0, cnt, body, 0)

        @pl.when(lax.rem(cnt, 2) == 0)
        def _():
            wait_out(0, ws0)
            wait_out(1, ws1)

        @pl.when(lax.rem(cnt, 2) == 1)
        def _():
            wait_out(0, ws0)

            @pl.when(cnt >= 2)
            def _():
                wait_out(1, ws1)

        @pl.when(wid == NW - 1)
        def _():
            pltpu.sync_copy(tail_hbm, tr_v.at[0, pl.ds(0, 16)])
            pltpu.sync_copy(
                tr_v.at[0, pl.ds(0, 16)], lin_hbm.at[pl.ds(NTC * 32, 16)]
            )

    @functools.partial(
        pl.kernel,
        mesh=mesh,
        out_type=jax.ShapeDtypeStruct((HIST, 4, NW, 8, 128), jnp.float32),
        scratch_types=[
            pltpu.VMEM((25, 8, 128), jnp.int32),
            pltpu.VMEM((2, 256, 32), jnp.float32),
            pltpu.VMEM((2, 2, 4, 8, 128), jnp.float32),
            pltpu.SemaphoreType.DMA,
            pltpu.SemaphoreType.DMA,
            pltpu.SemaphoreType.DMA,
            pltpu.SemaphoreType.DMA,
        ],
        compiler_params=pltpu.CompilerParams(
            use_tc_tiling_on_sc=False, needs_layout_passes=False
        ),
    )
    def gather_k(idx_hbm, lin_hbm, out_hbm, idx_v, rw_v, tr_v, gs0, gs1, ws0, ws1):
        wid = lax.axis_index("s") * NC + lax.axis_index("c")
        pltpu.sync_copy(idx_hbm.at[:, wid], idx_v)
        it16 = lax.iota(jnp.int32, 16)

        def fire_gather(jb, b, sem):
            for t in range(2):
                h = 2 * jb + t
                pltpu.async_copy(
                    lin_hbm.at[idx_v.at[h // 8, h % 8]],
                    rw_v.at[b, pl.ds(128 * t, 128)],
                    sem,
                )

        def wait_gather(b, sem):
            for t in range(2):
                pltpu.make_async_copy(
                    lin_hbm.at[idx_v.at[0, 0]],
                    rw_v.at[b, pl.ds(128 * t, 128)],
                    sem,
                ).wait()

        def do_transpose(b):
            src = rw_v.at[b]
            dst = tr_v.at[b]
            for t in range(2):
                for tr in range(4):
                    for sv in range(8):
                        for q in range(8):
                            row = it16 + (128 * t + 16 * q)
                            col = jnp.full((16,), 8 * tr + sv, jnp.int32)
                            vec = plsc.load_gather(src, [row, col])
                            plsc.store_scatter(
                                dst,
                                [
                                    jnp.full((16,), t, jnp.int32),
                                    jnp.full((16,), tr, jnp.int32),
                                    jnp.full((16,), sv, jnp.int32),
                                    it16 + 16 * q,
                                ],
                                vec,
                            )

        def fire_out(jb, b, sem):
            pltpu.async_copy(
                tr_v.at[b], out_hbm.at[pl.ds(2 * jb, 2), :, wid], sem
            )

        def wait_out(b, sem):
            pltpu.make_async_copy(
                tr_v.at[b], out_hbm.at[pl.ds(0, 2), :, wid], sem
            ).wait()

        fire_gather(0, 0, gs0)

        def body(jb, carry):
            b = lax.rem(jb, 2)
            nb = 1 - b

            @pl.when(jb + 1 < NHB)
            def _():
                @pl.when(b == 0)
                def _():
                    fire_gather(jb + 1, nb, gs1)

                @pl.when(b == 1)
                def _():
                    fire_gather(jb + 1, nb, gs0)

            @pl.when(b == 0)
            def _():
                wait_gather(b, gs0)

            @pl.when(b == 1)
            def _():
                wait_gather(b, gs1)

            @pl.when(jb >= 2)
            def _():
                @pl.when(b == 0)
                def _():
                    wait_out(b, ws0)

                @pl.when(b == 1)
                def _():
                    wait_out(b, ws1)

            do_transpose(b)

            @pl.when(b == 0)
            def _():
                fire_out(jb, b, ws0)

            @pl.when(b == 1)
            def _():
                fire_out(jb, b, ws1)

            return carry

        lax.fori_loop(0, NHB, body, 0)
        wait_out(0, ws0)
        wait_out(1, ws1)

    tail_lin = table[NTC * 128 :].reshape(16, 128)
    lin = transpose_k(table.T, tail_lin)
    idx_view = (
        tag_ids.astype(jnp.int32).T.reshape(25, 8, NW, 128).transpose(0, 2, 1, 3)
    )
    o = gather_k(idx_view, lin.reshape(VOCAB, 32))
    return o.transpose(2, 4, 0, 1, 3).reshape(BATCH, HIST, D)


def kernel(tag_ids, embedding_table):
    return _impl(tag_ids, embedding_table)


# speedup vs baseline: 1.8395x; 1.4139x over previous
import functools

import jax
import jax.numpy as jnp
from jax import lax
from jax.experimental import pallas as pl
from jax.experimental.pallas import tpu as pltpu
from jax.experimental.pallas import tpu_sc as plsc

BATCH = 4096
HIST = 200
D = 32
VOCAB = 1000000

NC = 2
NS = 16
NW = NC * NS
NTC = VOCAB // 128
LIN_ROWS = VOCAB // 4

NG = NTC // 2
GQ, GR = divmod(NG, NW)

NHB = HIST // 2


@jax.jit
def _impl(tag_ids, table):
    mesh = plsc.VectorSubcoreMesh(core_axis_name="c", subcore_axis_name="s")

    @functools.partial(
        pl.kernel,
        mesh=mesh,
        out_type=jax.ShapeDtypeStruct((LIN_ROWS, 128), jnp.float32),
        scratch_types=[
            pltpu.VMEM((2, 64, 128), jnp.float32),
            pltpu.VMEM((2, 64, 128), jnp.float32),
            pltpu.SemaphoreType.DMA,
            pltpu.SemaphoreType.DMA,
            pltpu.SemaphoreType.DMA,
            pltpu.SemaphoreType.DMA,
        ],
        compiler_params=pltpu.CompilerParams(
            use_tc_tiling_on_sc=True, needs_layout_passes=False
        ),
    )
    def transpose_k(table_hbm, tail_hbm, lin_hbm, in_v, tr_v, gs0, gs1, ws0, ws1):
        wid = lax.axis_index("s") * NC + lax.axis_index("c")
        start = jnp.where(wid < GR, wid * (GQ + 1), GR * (GQ + 1) + (wid - GR) * GQ)
        cnt = jnp.where(wid < GR, GQ + 1, GQ)
        it16 = lax.iota(jnp.int32, 16)

        def fire_in(j, b, sem):
            tc = (start + j) * 2
            pltpu.async_copy(
                table_hbm.at[:, pl.ds(tc * 128, 128)],
                in_v.at[b, pl.ds(0, 32)], sem,
            )
            pltpu.async_copy(
                table_hbm.at[:, pl.ds((tc + 1) * 128, 128)],
                in_v.at[b, pl.ds(32, 32)], sem,
            )

        def wait_in(b, sem):
            for half in range(2):
                pltpu.make_async_copy(
                    table_hbm.at[:, pl.ds(0, 128)],
                    in_v.at[b, pl.ds(32 * half, 32)], sem,
                ).wait()

        def do_transpose(b):
            src = in_v.at[b]
            dst = tr_v.at[b]
            for i in range(64):
                vecs = []
                for q in range(8):
                    rho = 4 * i + q // 2
                    row = it16 + (32 * (rho // 128) + 16 * (q % 2))
                    col = jnp.full((16,), rho % 128, jnp.int32)
                    vecs.append(plsc.load_gather(src, [row, col]))
                for q in range(8):
                    plsc.store_scatter(
                        dst,
                        [jnp.full((16,), i, jnp.int32), it16 + 16 * q],
                        vecs[q],
                    )

        def fire_out(j, b, sem):
            pltpu.async_copy(
                tr_v.at[b], lin_hbm.at[pl.ds((start + j) * 64, 64)], sem
            )

        def wait_out(b, sem):
            pltpu.make_async_copy(
                tr_v.at[b], lin_hbm.at[pl.ds(0, 64)], sem
            ).wait()

        fire_in(0, 0, gs0)

        def body(j, carry):
            b = lax.rem(j, 2)
            nb = 1 - b

            @pl.when(j + 1 < cnt)
            def _():
                @pl.when(b == 0)
                def _():
                    fire_in(j + 1, nb, gs1)

                @pl.when(b == 1)
                def _():
                    fire_in(j + 1, nb, gs0)

            @pl.when(b == 0)
            def _():
                wait_in(b, gs0)

            @pl.when(b == 1)
            def _():
                wait_in(b, gs1)

            @pl.when(j >= 2)
            def _():
                @pl.when(b == 0)
                def _():
                    wait_out(b, ws0)

                @pl.when(b == 1)
                def _():
                    wait_out(b, ws1)

            do_transpose(b)

            @pl.when(b == 0)
            def _():
                fire_out(j, b, ws0)

            @pl.when(b == 1)
            def _():
                fire_out(j, b, ws1)

            return carry

        lax.fori_loop(0, cnt, body, 0)

        @pl.when(lax.rem(cnt, 2) == 0)
        def _():
            wait_out(0, ws0)
            wait_out(1, ws1)

        @pl.when(lax.rem(cnt, 2) == 1)
        def _():
            wait_out(0, ws0)

            @pl.when(cnt >= 2)
            def _():
                wait_out(1, ws1)

        @pl.when(wid == NW - 1)
        def _():
            pltpu.sync_copy(tail_hbm, tr_v.at[0, pl.ds(0, 16)])
            pltpu.sync_copy(
                tr_v.at[0, pl.ds(0, 16)], lin_hbm.at[pl.ds(NTC * 32, 16)]
            )

    @functools.partial(
        pl.kernel,
        mesh=mesh,
        out_type=jax.ShapeDtypeStruct((HIST, 4, NW, 8, 128), jnp.float32),
        scratch_types=[
            pltpu.VMEM((25, 8, 128), jnp.int32),
            pltpu.VMEM((2, 256, 32), jnp.float32),
            pltpu.VMEM((2, 2, 4, 8, 128), jnp.float32),
            pltpu.SemaphoreType.DMA,
            pltpu.SemaphoreType.DMA,
            pltpu.SemaphoreType.DMA,
            pltpu.SemaphoreType.DMA,
        ],
        compiler_params=pltpu.CompilerParams(
            use_tc_tiling_on_sc=False, needs_layout_passes=False
        ),
    )
    def gather_k(idx_hbm, lin_hbm, out_hbm, idx_v, rw_v, tr_v, gs0, gs1, ws0, ws1):
        wid = lax.axis_index("s") * NC + lax.axis_index("c")
        pltpu.sync_copy(idx_hbm.at[:, wid], idx_v)
        it16 = lax.iota(jnp.int32, 16)

        def fire_gather(jb, b, sem):
            for t in range(2):
                h = 2 * jb + t
                pltpu.async_copy(
                    lin_hbm.at[idx_v.at[h // 8, h % 8]],
                    rw_v.at[b, pl.ds(128 * t, 128)],
                    sem,
                )

        def wait_gather(b, sem):
            for t in range(2):
                pltpu.make_async_copy(
                    lin_hbm.at[idx_v.at[0, 0]],
                    rw_v.at[b, pl.ds(128 * t, 128)],
                    sem,
                ).wait()

        def do_transpose(b):
            src = rw_v.at[b]
            dst = tr_v.at[b]
            for t in range(2):
                for tr in range(4):
                    for sv in range(8):
                        vecs = []
                        for q in range(8):
                            row = it16 + (128 * t + 16 * q)
                            col = jnp.full((16,), 8 * tr + sv, jnp.int32)
                            vecs.append(plsc.load_gather(src, [row, col]))
                        for q in range(8):
                            plsc.store_scatter(
                                dst,
                                [
                                    jnp.full((16,), t, jnp.int32),
                                    jnp.full((16,), tr, jnp.int32),
                                    jnp.full((16,), sv, jnp.int32),
                                    it16 + 16 * q,
                                ],
                                vecs[q],
                            )

        def fire_out(jb, b, sem):
            pltpu.async_copy(
                tr_v.at[b], out_hbm.at[pl.ds(2 * jb, 2), :, wid], sem
            )

        def wait_out(b, sem):
            pltpu.make_async_copy(
                tr_v.at[b], out_hbm.at[pl.ds(0, 2), :, wid], sem
            ).wait()

        fire_gather(0, 0, gs0)

        def body(jb, carry):
            b = lax.rem(jb, 2)
            nb = 1 - b

            @pl.when(jb + 1 < NHB)
            def _():
                @pl.when(b == 0)
                def _():
                    fire_gather(jb + 1, nb, gs1)

                @pl.when(b == 1)
                def _():
                    fire_gather(jb + 1, nb, gs0)

            @pl.when(b == 0)
            def _():
                wait_gather(b, gs0)

            @pl.when(b == 1)
            def _():
                wait_gather(b, gs1)

            @pl.when(jb >= 2)
            def _():
                @pl.when(b == 0)
                def _():
                    wait_out(b, ws0)

                @pl.when(b == 1)
                def _():
                    wait_out(b, ws1)

            do_transpose(b)

            @pl.when(b == 0)
            def _():
                fire_out(jb, b, ws0)

            @pl.when(b == 1)
            def _():
                fire_out(jb, b, ws1)

            return carry

        lax.fori_loop(0, NHB, body, 0)
        wait_out(0, ws0)
        wait_out(1, ws1)

    tail_lin = table[NTC * 128 :].reshape(16, 128)
    lin = transpose_k(table.T, tail_lin)
    idx_view = (
        tag_ids.astype(jnp.int32).T.reshape(25, 8, NW, 128).transpose(0, 2, 1, 3)
    )
    o = gather_k(idx_view, lin.reshape(VOCAB, 32))
    return o.transpose(2, 4, 0, 1, 3).reshape(BATCH, HIST, D)


def kernel(tag_ids, embedding_table):
    return _impl(tag_ids, embedding_table)


# speedup vs baseline: 1.8717x; 1.0175x over previous
import functools

import jax
import jax.numpy as jnp
from jax import lax
from jax.experimental import pallas as pl
from jax.experimental.pallas import tpu as pltpu
from jax.experimental.pallas import tpu_sc as plsc

BATCH = 4096
HIST = 200
D = 32
TOTAL = BATCH * HIST

NC = 2
NS = 16
NW = NC * NS
ROWS_PER_W = TOTAL // NW
STREAM = 128
K = 10
CHUNK = STREAM * K
NCHUNK = ROWS_PER_W // CHUNK
NSTREAM = ROWS_PER_W // STREAM


@jax.jit
def _gather(tag_ids_r, table):
    mesh = plsc.VectorSubcoreMesh(core_axis_name="c", subcore_axis_name="s")

    @functools.partial(
        pl.kernel,
        mesh=mesh,
        out_type=jax.ShapeDtypeStruct((NW, NCHUNK, K, STREAM, D), jnp.float32),
        scratch_types=[
            pltpu.VMEM((NSTREAM, STREAM), jnp.int32),
            pltpu.VMEM((2, K, STREAM, D), jnp.float32),
            pltpu.SemaphoreType.DMA,
            pltpu.SemaphoreType.DMA,
            pltpu.SemaphoreType.DMA,
            pltpu.SemaphoreType.DMA,
        ],
        compiler_params=pltpu.CompilerParams(use_tc_tiling_on_sc=False),
    )
    def k(idx_hbm, table_hbm, out_hbm, idx_v, rows_v, gsem0, gsem1, wsem0, wsem1):
        wid = lax.axis_index("s") * NC + lax.axis_index("c")
        pltpu.sync_copy(idx_hbm.at[wid], idx_v)

        bufs = (rows_v.at[0], rows_v.at[1])
        gsems = (gsem0, gsem1)
        wsems = (wsem0, wsem1)

        def fire_gathers(c, i):
            for b in range(K):
                pltpu.async_copy(
                    table_hbm.at[idx_v.at[c * K + b]], bufs[i].at[b], gsems[i]
                )

        def drain_gathers(i):
            for b in range(K):
                pltpu.make_async_copy(
                    table_hbm.at[idx_v.at[b]], bufs[i].at[b], gsems[i]
                ).wait()

        def fire_write(c, i):
            pltpu.async_copy(bufs[i], out_hbm.at[wid, c], wsems[i])

        def drain_write(i):
            pltpu.make_async_copy(bufs[i], out_hbm.at[wid, 0], wsems[i]).wait()

        fire_gathers(0, 0)

        def body(c, carry):
            def step(i):
                nxt = 1 - i

                @pl.when(c >= 1)
                def _():
                    drain_write(nxt)

                @pl.when(c + 1 < NCHUNK)
                def _():
                    fire_gathers(c + 1, nxt)

                drain_gathers(i)
                fire_write(c, i)

            is_even = lax.rem(c, 2) == 0

            @pl.when(is_even)
            def _():
                step(0)

            @pl.when(jnp.logical_not(is_even))
            def _():
                step(1)

            return carry

        lax.fori_loop(0, NCHUNK, body, 0)
        drain_write((NCHUNK - 1) % 2)

    return k(tag_ids_r, table)


def kernel(tag_ids, embedding_table):
    idx = tag_ids.reshape(NW, NSTREAM, STREAM).astype(jnp.int32)
    out = _gather(idx, embedding_table)
    return out.reshape(BATCH, HIST, D)
